# Initial kernel scaffold; baseline (speedup 1.0000x reference)
import jax
import jax.numpy as jnp
from jax.experimental import pallas as pl


def kernel(x, edge_index, W1, b1, W2, b2):
    raise NotImplementedError("write your pallas kernel here")


# speedup vs baseline: 9.4182x; 9.4182x over previous
import functools

import jax
import jax.numpy as jnp
from jax import lax
from jax.experimental import pallas as pl
from jax.experimental.pallas import tpu as pltpu
from jax.experimental.pallas import tpu_sc as plsc

N = 10000
E = 320000
D = 128
NC = 2
NS = 16
NW = NC * NS
CHUNK = 128
EPW = -(-E // (NW * CHUNK)) * CHUNK
EPAD = EPW * NW
NP = 10240
DW = 128

_mesh = plsc.VectorSubcoreMesh(core_axis_name="c", subcore_axis_name="s")


@functools.partial(
    pl.kernel,
    out_type=jax.ShapeDtypeStruct((NC, NP, DW), jnp.float32),
    mesh=_mesh,
    scratch_types=[
        pltpu.VMEM_SHARED((NP, DW), jnp.float32),
        pltpu.VMEM((CHUNK,), jnp.int32),
        pltpu.VMEM((CHUNK, DW), jnp.float32),
    ],
)
def _deg_sc(col_hbm, ones_hbm, zeros_hbm, out_hbm, acc_sp, cidx, ones_v):
    c = lax.axis_index("c")
    s = lax.axis_index("s")
    wid = s * NC + c
    rz = NP // NS
    pltpu.sync_copy(zeros_hbm.at[pl.ds(s * rz, rz)], acc_sp.at[pl.ds(s * rz, rz)])
    pltpu.sync_copy(ones_hbm, ones_v)
    plsc.subcore_barrier()

    e0 = wid * EPW

    @pl.loop(0, EPW // CHUNK)
    def _(i):
        pltpu.sync_copy(col_hbm.at[pl.ds(e0 + i * CHUNK, CHUNK)], cidx)
        pltpu.sync_copy(ones_v, acc_sp.at[cidx], add=True)

    plsc.subcore_barrier()
    pltpu.sync_copy(acc_sp.at[pl.ds(s * rz, rz)], out_hbm.at[c, pl.ds(s * rz, rz)])


@functools.partial(
    pl.kernel,
    out_type=jax.ShapeDtypeStruct((NC, NP, D), jnp.float32),
    mesh=_mesh,
    scratch_types=[
        pltpu.VMEM_SHARED((NP, D), jnp.float32),
        pltpu.VMEM((CHUNK,), jnp.int32),
        pltpu.VMEM((CHUNK,), jnp.int32),
        pltpu.VMEM((CHUNK, D), jnp.float32),
        pltpu.SemaphoreType.DMA,
    ],
)
def _agg_sc(y_hbm, row_hbm, col_hbm, zeros_hbm, out_hbm,
            acc_sp, ridx, cidx, rows_v, sem):
    c = lax.axis_index("c")
    s = lax.axis_index("s")
    wid = s * NC + c
    rz = NP // NS
    pltpu.sync_copy(zeros_hbm.at[pl.ds(s * rz, rz)], acc_sp.at[pl.ds(s * rz, rz)])
    plsc.subcore_barrier()

    e0 = wid * EPW

    @pl.loop(0, EPW // CHUNK)
    def _(i):
        base = e0 + i * CHUNK
        pltpu.sync_copy(row_hbm.at[pl.ds(base, CHUNK)], ridx)
        pltpu.sync_copy(col_hbm.at[pl.ds(base, CHUNK)], cidx)
        pltpu.async_copy(y_hbm.at[ridx], rows_v, sem).wait()
        pltpu.sync_copy(rows_v, acc_sp.at[cidx], add=True)

    plsc.subcore_barrier()
    pltpu.sync_copy(acc_sp.at[pl.ds(s * rz, rz)], out_hbm.at[c, pl.ds(s * rz, rz)])


_BM = 1000


def _dvec(degp_ref):
    deg = degp_ref[0, :, 0] + degp_ref[1, :, 0] + 2.0
    return lax.rsqrt(deg)[:, None]


def _mm_scale(x, W, degp):
    def body(x_ref, w_ref, degp_ref, o_ref):
        d = _dvec(degp_ref)
        o_ref[...] = d * jnp.dot(x_ref[...], w_ref[...],
                                 preferred_element_type=jnp.float32)

    return pl.pallas_call(
        body,
        grid=(N // _BM,),
        in_specs=[
            pl.BlockSpec((_BM, D), lambda i: (i, 0)),
            pl.BlockSpec((D, D), lambda i: (0, 0)),
            pl.BlockSpec((NC, _BM, DW), lambda i: (0, i, 0)),
        ],
        out_specs=pl.BlockSpec((_BM, D), lambda i: (i, 0)),
        out_shape=jax.ShapeDtypeStruct((N, D), jnp.float32),
    )(x, W, degp)


def _mid(aggp, y1, degp, b1, W2):
    def body(a_ref, y_ref, degp_ref, b_ref, w_ref, o_ref):
        d = _dvec(degp_ref)
        h = d * (a_ref[0] + a_ref[1] + 2.0 * y_ref[...]) + b_ref[...]
        h = jnp.maximum(h, 0.0)
        o_ref[...] = d * jnp.dot(h, w_ref[...],
                                 preferred_element_type=jnp.float32)

    return pl.pallas_call(
        body,
        grid=(N // _BM,),
        in_specs=[
            pl.BlockSpec((NC, _BM, D), lambda i: (0, i, 0)),
            pl.BlockSpec((_BM, D), lambda i: (i, 0)),
            pl.BlockSpec((NC, _BM, DW), lambda i: (0, i, 0)),
            pl.BlockSpec((1, D), lambda i: (0, 0)),
            pl.BlockSpec((D, D), lambda i: (0, 0)),
        ],
        out_specs=pl.BlockSpec((_BM, D), lambda i: (i, 0)),
        out_shape=jax.ShapeDtypeStruct((N, D), jnp.float32),
    )(aggp, y1, degp, b1, W2)


def _post(aggp, y2, degp, b2):
    def body(a_ref, y_ref, degp_ref, b_ref, o_ref):
        d = _dvec(degp_ref)
        o_ref[...] = d * (a_ref[0] + a_ref[1] + 2.0 * y_ref[...]) + b_ref[...]

    return pl.pallas_call(
        body,
        grid=(N // _BM,),
        in_specs=[
            pl.BlockSpec((NC, _BM, D), lambda i: (0, i, 0)),
            pl.BlockSpec((_BM, D), lambda i: (i, 0)),
            pl.BlockSpec((NC, _BM, DW), lambda i: (0, i, 0)),
            pl.BlockSpec((1, D), lambda i: (0, 0)),
        ],
        out_specs=pl.BlockSpec((_BM, D), lambda i: (i, 0)),
        out_shape=jax.ShapeDtypeStruct((N, D), jnp.float32),
    )(aggp, y2, degp, b2)


def kernel(x, edge_index, W1, b1, W2, b2):
    ei = edge_index.astype(jnp.int32)
    row = jnp.concatenate([ei[0], jnp.zeros((EPAD - E,), jnp.int32)])
    col = jnp.concatenate([ei[1], jnp.full((EPAD - E,), N, jnp.int32)])
    zeros_d = jnp.zeros((NP, D), jnp.float32)
    zeros_w = jnp.zeros((NP, DW), jnp.float32)
    ones_w = jnp.ones((CHUNK, DW), jnp.float32)

    degp = _deg_sc(col, ones_w, zeros_w)
    y1 = _mm_scale(x, W1, degp)
    aggp1 = _agg_sc(y1, row, col, zeros_d)
    y2 = _mid(aggp1, y1, degp, b1.reshape(1, D), W2)
    aggp2 = _agg_sc(y2, row, col, zeros_d)
    return _post(aggp2, y2, degp, b2.reshape(1, D))
